# baseline (speedup 1.0000x reference)
import functools

import jax
import jax.numpy as jnp
from jax import lax
from jax.experimental import pallas as pl
from jax.experimental.pallas import tpu as pltpu, tpu_sc as plsc

_DIM = 256
_LANES = 16


def _sc_row_lookup(idx, table_flat):
    mesh = plsc.VectorSubcoreMesh(
        core_axis_name="c", subcore_axis_name="s", num_cores=1, num_subcores=1
    )

    @functools.partial(
        pl.kernel,
        mesh=mesh,
        out_type=jax.ShapeDtypeStruct((_DIM,), jnp.float32),
        compiler_params=pltpu.CompilerParams(
            skip_device_barrier=True,
            disable_bounds_checks=True,
            disable_semaphore_checks=True,
        ),
        scratch_types=[
            pltpu.VMEM((_LANES,), jnp.int32),
            pltpu.VMEM((4 * _DIM,), jnp.float32),
            pltpu.VMEM((_DIM,), jnp.float32),
            pltpu.SemaphoreType.DMA,
            pltpu.SemaphoreType.DMA,
        ],
    )
    def k(idx_hbm, table_hbm, out_hbm, idx_v, tab_v, row_v, sem1, sem2):
        cp_idx = pltpu.async_copy(idx_hbm, idx_v, sem1)
        cp_tab = pltpu.async_copy(table_hbm, tab_v, sem2)
        cp_idx.wait()
        sel = idx_v[...]
        cp_tab.wait()
        for j in range(_DIM // _LANES):
            off = _LANES * j
            v = tab_v[pl.ds(3 * _DIM + off, _LANES)]
            for r in (2, 1, 0):
                v = jnp.where(sel == r, tab_v[pl.ds(r * _DIM + off, _LANES)], v)
            row_v[pl.ds(off, _LANES)] = v
        pltpu.sync_copy(row_v, out_hbm)

    return k(idx, table_flat)


def kernel(tensor_list, scale, scale_embed_weight):
    idx = jnp.full((_LANES,), jnp.asarray(scale, jnp.int32), jnp.int32)
    row = _sc_row_lookup(idx, scale_embed_weight.reshape((-1,)))
    return row.reshape((1, _DIM, 1, 1))


# speedup vs baseline: 1.0209x; 1.0209x over previous
import functools

import jax
import jax.numpy as jnp
from jax.experimental import pallas as pl
from jax.experimental.pallas import tpu as pltpu, tpu_sc as plsc

_DIM = 256
_LANES = 16


def _sc_row_lookup(idx, table_flat):
    mesh = plsc.VectorSubcoreMesh(
        core_axis_name="c", subcore_axis_name="s", num_cores=1, num_subcores=1
    )

    @functools.partial(
        pl.kernel,
        mesh=mesh,
        out_type=jax.ShapeDtypeStruct((_DIM,), jnp.float32),
        scratch_types=[
            pltpu.VMEM((_LANES,), jnp.int32),
            pltpu.VMEM((4 * _DIM,), jnp.float32),
            pltpu.VMEM((_DIM,), jnp.float32),
            pltpu.SemaphoreType.DMA,
            pltpu.SemaphoreType.DMA,
        ],
    )
    def k(idx_hbm, table_hbm, out_hbm, idx_v, tab_v, row_v, sem1, sem2):
        cp_idx = pltpu.async_copy(idx_hbm, idx_v, sem1)
        cp_tab = pltpu.async_copy(table_hbm, tab_v, sem2)
        cp_idx.wait()
        sel = idx_v[...]
        cp_tab.wait()
        for j in range(_DIM // _LANES):
            off = _LANES * j
            v = tab_v[pl.ds(3 * _DIM + off, _LANES)]
            for r in (2, 1, 0):
                v = jnp.where(sel == r, tab_v[pl.ds(r * _DIM + off, _LANES)], v)
            row_v[pl.ds(off, _LANES)] = v
        pltpu.sync_copy(row_v, out_hbm)

    return k(idx, table_flat)


def kernel(tensor_list, scale, scale_embed_weight):
    idx = jnp.full((_LANES,), jnp.asarray(scale, jnp.int32), jnp.int32)
    row = _sc_row_lookup(idx, scale_embed_weight.reshape((-1,)))
    return row.reshape((1, _DIM, 1, 1))
